# baseline (speedup 1.0000x reference)
import functools

import jax
import jax.numpy as jnp
from jax import lax
from jax.experimental import pallas as pl
from jax.experimental.pallas import tpu as pltpu
from jax.experimental.pallas import tpu_sc as plsc

ROWS = 65536
C = 313
LANES = 16
NW = 32
SC_ROWS = 40960
RPW = SC_ROWS // NW
CHUNK_ROWS = 80
GROUPS = CHUNK_ROWS // LANES
NCHUNK = RPW // CHUNK_ROWS
NEG = -3.4e38


def _compute_chunk(ob, bb, acc):
    lanes = lax.iota(jnp.int32, LANES)
    rows = [lanes + jnp.int32(g * LANES) for g in range(GROUPS)]

    neg = jnp.full((LANES,), NEG, jnp.float32)
    init = (lanes, (neg,) * GROUPS, (neg,) * GROUPS, (neg,) * GROUPS)

    @plsc.parallel_loop(0, C, carry=init, unroll=4)
    def step(c, st):
        cols, bbest, obest, omax = st
        nb, no, nm = [], [], []
        for g in range(GROUPS):
            bv = plsc.load_gather(bb, [rows[g], cols])
            ov = plsc.load_gather(ob, [rows[g], cols])
            better = bv > bbest[g]
            nb.append(jnp.where(better, bv, bbest[g]))
            no.append(jnp.where(better, ov, obest[g]))
            nm.append(jnp.maximum(omax[g], ov))
        ncols = cols + 1
        ncols = jnp.where(ncols >= C, ncols - C, ncols)
        return (ncols, tuple(nb), tuple(no), tuple(nm))

    _, _, obest, omax = step
    for g in range(GROUPS):
        acc = acc + (1.0 - obest[g] / omax[g])
    return acc


@functools.partial(
    pl.kernel,
    out_type=jax.ShapeDtypeStruct((NW, LANES), jnp.float32),
    mesh=plsc.VectorSubcoreMesh(core_axis_name="c", subcore_axis_name="s"),
    compiler_params=pltpu.CompilerParams(needs_layout_passes=False),
    scratch_types=[
        pltpu.VMEM((CHUNK_ROWS, C), jnp.float32),
        pltpu.VMEM((CHUNK_ROWS, C), jnp.float32),
        pltpu.VMEM((CHUNK_ROWS, C), jnp.float32),
        pltpu.VMEM((CHUNK_ROWS, C), jnp.float32),
        pltpu.VMEM((LANES,), jnp.float32),
        pltpu.SemaphoreType.DMA,
        pltpu.SemaphoreType.DMA,
        pltpu.SemaphoreType.DMA,
        pltpu.SemaphoreType.DMA,
    ],
)
def _sc_loss(orig_hbm, boost_hbm, out_hbm,
             ob0, bb0, ob1, bb1, accbuf, so0, sb0, so1, sb1):
    wid = lax.axis_index("s") * 2 + lax.axis_index("c")
    wbase = wid * RPW

    def start(ob, bb, so, sb, g):
        row0 = wbase + g * CHUNK_ROWS
        pltpu.async_copy(orig_hbm.at[pl.ds(row0, CHUNK_ROWS), :], ob, so)
        pltpu.async_copy(boost_hbm.at[pl.ds(row0, CHUNK_ROWS), :], bb, sb)

    def wait(ob, bb, so, sb, g):
        row0 = wbase + g * CHUNK_ROWS
        pltpu.make_async_copy(
            orig_hbm.at[pl.ds(row0, CHUNK_ROWS), :], ob, so).wait()
        pltpu.make_async_copy(
            boost_hbm.at[pl.ds(row0, CHUNK_ROWS), :], bb, sb).wait()

    start(ob0, bb0, so0, sb0, 0)

    def outer(i, acc):
        g0 = 2 * i
        wait(ob0, bb0, so0, sb0, g0)
        start(ob1, bb1, so1, sb1, g0 + 1)
        acc = _compute_chunk(ob0, bb0, acc)
        wait(ob1, bb1, so1, sb1, g0 + 1)

        @pl.when(i < NCHUNK // 2 - 1)
        def _():
            start(ob0, bb0, so0, sb0, g0 + 2)

        return _compute_chunk(ob1, bb1, acc)

    acc = lax.fori_loop(0, NCHUNK // 2, outer, jnp.zeros((LANES,), jnp.float32))
    accbuf[...] = acc
    pltpu.sync_copy(accbuf, out_hbm.at[wid])


TC_BLOCK = 1024
TC_START = SC_ROWS
TC_GRID = (ROWS - TC_START) // TC_BLOCK


def _tc_body(orig_ref, boost_ref, out_ref):
    b = boost_ref[...]
    o = orig_ref[...]
    m = jnp.max(b, axis=1, keepdims=True)
    lookup = jnp.max(jnp.where(b == m, o, NEG), axis=1)
    omax = jnp.max(o, axis=1)
    out_ref[...] = jnp.sum(1.0 - lookup / omax).reshape(1, 1, 1)


_tc_loss = pl.pallas_call(
    _tc_body,
    grid=(TC_GRID,),
    in_specs=[
        pl.BlockSpec((TC_BLOCK, C), lambda i: (i + TC_START // TC_BLOCK, 0)),
        pl.BlockSpec((TC_BLOCK, C), lambda i: (i + TC_START // TC_BLOCK, 0)),
    ],
    out_specs=pl.BlockSpec((1, 1, 1), lambda i: (i, 0, 0)),
    out_shape=jax.ShapeDtypeStruct((TC_GRID, 1, 1), jnp.float32),
)


def kernel(original, boosted):
    orig = original.reshape(ROWS, C)
    boost = boosted.reshape(ROWS, C)
    partials_sc = _sc_loss(orig, boost)
    partials_tc = _tc_loss(orig, boost)
    return jnp.sum(partials_sc) + jnp.sum(partials_tc)


# speedup vs baseline: 1.0363x; 1.0363x over previous
import functools

import jax
import jax.numpy as jnp
from jax import lax
from jax.experimental import pallas as pl
from jax.experimental.pallas import tpu as pltpu
from jax.experimental.pallas import tpu_sc as plsc

ROWS = 65536
C = 313
LANES = 16
NW = 32
SC_ROWS = 36864
RPW = SC_ROWS // NW
CHUNK_ROWS = 64
GROUPS = CHUNK_ROWS // LANES
NCHUNK = RPW // CHUNK_ROWS
NEG = -3.4e38


def _compute_chunk(ob, bb, acc):
    lanes = lax.iota(jnp.int32, LANES)
    rows = [lanes + jnp.int32(g * LANES) for g in range(GROUPS)]

    neg = jnp.full((LANES,), NEG, jnp.float32)
    init = (lanes, (neg,) * GROUPS, (neg,) * GROUPS, (neg,) * GROUPS)

    @plsc.parallel_loop(0, C, carry=init, unroll=4)
    def step(c, st):
        cols, bbest, obest, omax = st
        nb, no, nm = [], [], []
        for g in range(GROUPS):
            bv = plsc.load_gather(bb, [rows[g], cols])
            ov = plsc.load_gather(ob, [rows[g], cols])
            better = bv > bbest[g]
            nb.append(jnp.where(better, bv, bbest[g]))
            no.append(jnp.where(better, ov, obest[g]))
            nm.append(jnp.maximum(omax[g], ov))
        ncols = cols + 1
        ncols = jnp.where(ncols >= C, ncols - C, ncols)
        return (ncols, tuple(nb), tuple(no), tuple(nm))

    _, _, obest, omax = step
    for g in range(GROUPS):
        acc = acc + (1.0 - obest[g] / omax[g])
    return acc


@functools.partial(
    pl.kernel,
    out_type=jax.ShapeDtypeStruct((NW, LANES), jnp.float32),
    mesh=plsc.VectorSubcoreMesh(core_axis_name="c", subcore_axis_name="s"),
    compiler_params=pltpu.CompilerParams(needs_layout_passes=False),
    scratch_types=[
        pltpu.VMEM((CHUNK_ROWS, C), jnp.float32),
        pltpu.VMEM((CHUNK_ROWS, C), jnp.float32),
        pltpu.VMEM((CHUNK_ROWS, C), jnp.float32),
        pltpu.VMEM((CHUNK_ROWS, C), jnp.float32),
        pltpu.VMEM((LANES,), jnp.float32),
        pltpu.SemaphoreType.DMA,
        pltpu.SemaphoreType.DMA,
        pltpu.SemaphoreType.DMA,
        pltpu.SemaphoreType.DMA,
    ],
)
def _sc_loss(orig_hbm, boost_hbm, out_hbm,
             ob0, bb0, ob1, bb1, accbuf, so0, sb0, so1, sb1):
    wid = lax.axis_index("s") * 2 + lax.axis_index("c")
    wbase = wid * RPW

    def start(ob, bb, so, sb, g):
        row0 = wbase + g * CHUNK_ROWS
        pltpu.async_copy(orig_hbm.at[pl.ds(row0, CHUNK_ROWS), :], ob, so)
        pltpu.async_copy(boost_hbm.at[pl.ds(row0, CHUNK_ROWS), :], bb, sb)

    def wait(ob, bb, so, sb, g):
        row0 = wbase + g * CHUNK_ROWS
        pltpu.make_async_copy(
            orig_hbm.at[pl.ds(row0, CHUNK_ROWS), :], ob, so).wait()
        pltpu.make_async_copy(
            boost_hbm.at[pl.ds(row0, CHUNK_ROWS), :], bb, sb).wait()

    start(ob0, bb0, so0, sb0, 0)

    def outer(i, acc):
        g0 = 2 * i
        wait(ob0, bb0, so0, sb0, g0)
        start(ob1, bb1, so1, sb1, g0 + 1)
        acc = _compute_chunk(ob0, bb0, acc)
        wait(ob1, bb1, so1, sb1, g0 + 1)

        @pl.when(i < NCHUNK // 2 - 1)
        def _():
            start(ob0, bb0, so0, sb0, g0 + 2)

        return _compute_chunk(ob1, bb1, acc)

    acc = lax.fori_loop(0, NCHUNK // 2, outer, jnp.zeros((LANES,), jnp.float32))
    accbuf[...] = acc
    pltpu.sync_copy(accbuf, out_hbm.at[wid])


TC_BLOCK = 1024
TC_START = SC_ROWS
TC_GRID = (ROWS - TC_START) // TC_BLOCK


def _tc_body(orig_ref, boost_ref, out_ref):
    b = boost_ref[...]
    o = orig_ref[...]
    m = jnp.max(b, axis=1, keepdims=True)
    lookup = jnp.max(jnp.where(b == m, o, NEG), axis=1)
    omax = jnp.max(o, axis=1)
    out_ref[...] = jnp.sum(1.0 - lookup / omax).reshape(1, 1, 1)


_tc_loss = pl.pallas_call(
    _tc_body,
    grid=(TC_GRID,),
    in_specs=[
        pl.BlockSpec((TC_BLOCK, C), lambda i: (i + TC_START // TC_BLOCK, 0)),
        pl.BlockSpec((TC_BLOCK, C), lambda i: (i + TC_START // TC_BLOCK, 0)),
    ],
    out_specs=pl.BlockSpec((1, 1, 1), lambda i: (i, 0, 0)),
    out_shape=jax.ShapeDtypeStruct((TC_GRID, 1, 1), jnp.float32),
)


def kernel(original, boosted):
    orig = original.reshape(ROWS, C)
    boost = boosted.reshape(ROWS, C)
    partials_sc = _sc_loss(orig, boost)
    partials_tc = _tc_loss(orig, boost)
    return jnp.sum(partials_sc) + jnp.sum(partials_tc)
